# baseline (speedup 1.0000x reference)
import functools

import jax
import jax.numpy as jnp
from jax import lax
from jax.experimental import pallas as pl
from jax.experimental.pallas import tpu as pltpu
from jax.experimental.pallas import tpu_sc as plsc

EMBED = 64
_LANES_PER_STREAM = 128
_CHUNK = 1024
_STREAMS = _CHUNK // _LANES_PER_STREAM


@functools.lru_cache(maxsize=None)
def _make_gather(vocab: int, n_idx: int):
    info = plsc.get_sparse_core_info()
    nc, ns = info.num_cores, info.num_subcores
    nw = nc * ns
    assert n_idx % (nw * _CHUNK) == 0
    per_w = n_idx // nw
    n_chunks = per_w // _CHUNK
    idx_rows_per_chunk = _CHUNK // _LANES_PER_STREAM

    mesh = plsc.VectorSubcoreMesh(core_axis_name="c", subcore_axis_name="s")

    @functools.partial(
        pl.kernel,
        mesh=mesh,
        out_type=jax.ShapeDtypeStruct((n_idx, EMBED), jnp.float32),
        scratch_types=[
            pltpu.VMEM((idx_rows_per_chunk, _LANES_PER_STREAM), jnp.int32),
            pltpu.VMEM((_CHUNK, EMBED), jnp.float32),
            pltpu.SemaphoreType.DMA,
        ],
        compiler_params=pltpu.CompilerParams(use_tc_tiling_on_sc=False),
    )
    def gather(table_hbm, idx_hbm, out_hbm, idx_v, rows_v, sem):
        wid = lax.axis_index("s") * nc + lax.axis_index("c")
        idx_row0 = wid * (per_w // _LANES_PER_STREAM)
        out_base = wid * per_w

        def step(g, carry):
            pltpu.sync_copy(
                idx_hbm.at[pl.ds(idx_row0 + g * idx_rows_per_chunk,
                                 idx_rows_per_chunk)],
                idx_v,
            )
            handles = []
            for j in range(_STREAMS):
                handles.append(
                    pltpu.async_copy(
                        table_hbm.at[idx_v.at[j]],
                        rows_v.at[pl.ds(j * _LANES_PER_STREAM,
                                        _LANES_PER_STREAM)],
                        sem,
                    )
                )
            for h in handles:
                h.wait()
            pltpu.sync_copy(
                rows_v,
                out_hbm.at[pl.ds(out_base + g * _CHUNK, _CHUNK)],
            )
            return carry

        lax.fori_loop(0, n_chunks, step, 0)

    return gather


def kernel(x, word_embeddings):
    n_idx = x.size
    idx2d = x.reshape(-1, _LANES_PER_STREAM).astype(jnp.int32)
    out = _make_gather(word_embeddings.shape[0], n_idx)(word_embeddings, idx2d)
    return out.reshape(x.shape + (EMBED,))


# speedup vs baseline: 1.0178x; 1.0178x over previous
import functools

import jax
import jax.numpy as jnp
from jax import lax
from jax.experimental import pallas as pl
from jax.experimental.pallas import tpu as pltpu
from jax.experimental.pallas import tpu_sc as plsc

EMBED = 64
_IDXW = 128
_CHUNK = 512
_STREAMS = _CHUNK // _IDXW
_NSLOT = 3


@functools.lru_cache(maxsize=None)
def _make_gather(vocab: int, n_idx: int):
    info = plsc.get_sparse_core_info()
    nc, ns = info.num_cores, info.num_subcores
    nw = nc * ns
    assert n_idx % (nw * _CHUNK) == 0
    per_w = n_idx // nw
    n_chunks = per_w // _CHUNK
    idx_rows = per_w // _IDXW

    mesh = plsc.VectorSubcoreMesh(core_axis_name="c", subcore_axis_name="s")

    @functools.partial(
        pl.kernel,
        mesh=mesh,
        out_type=jax.ShapeDtypeStruct((n_idx, EMBED), jnp.float32),
        scratch_types=[
            pltpu.VMEM((idx_rows, _IDXW), jnp.int32),
            pltpu.VMEM((_NSLOT * _CHUNK, EMBED), jnp.float32),
            pltpu.SemaphoreType.DMA,
            pltpu.SemaphoreType.DMA,
        ],
        compiler_params=pltpu.CompilerParams(use_tc_tiling_on_sc=False),
    )
    def gather(table_hbm, idx_hbm, out_hbm, idx_v, rows_v, g_sem, w_sem):
        wid = lax.axis_index("s") * nc + lax.axis_index("c")
        out_base = wid * per_w

        pltpu.sync_copy(idx_hbm.at[pl.ds(wid * idx_rows, idx_rows)], idx_v)

        def fire_gathers(g, slot):
            for j in range(_STREAMS):
                pltpu.async_copy(
                    table_hbm.at[idx_v.at[g * _STREAMS + j]],
                    rows_v.at[pl.ds(slot * _CHUNK + j * _IDXW, _IDXW)],
                    g_sem,
                )

        def drain_gathers():
            pltpu.make_async_copy(
                table_hbm.at[pl.ds(0, _CHUNK)],
                rows_v.at[pl.ds(0, _CHUNK)],
                g_sem,
            ).wait()

        def drain_write():
            pltpu.make_async_copy(
                rows_v.at[pl.ds(0, _CHUNK)],
                out_hbm.at[pl.ds(out_base, _CHUNK)],
                w_sem,
            ).wait()

        fire_gathers(0, 0)

        def step(g, carry):
            nxt_slot = lax.rem(g + 1, _NSLOT)

            @pl.when(g >= _NSLOT - 1)
            def _():
                drain_write()

            @pl.when(g + 1 < n_chunks)
            def _():
                fire_gathers(g + 1, nxt_slot)

            drain_gathers()
            pltpu.async_copy(
                rows_v.at[pl.ds(lax.rem(g, _NSLOT) * _CHUNK, _CHUNK)],
                out_hbm.at[pl.ds(out_base + g * _CHUNK, _CHUNK)],
                w_sem,
            )
            return carry

        lax.fori_loop(0, n_chunks, step, 0)
        for _ in range(min(_NSLOT - 1, n_chunks)):
            drain_write()

    return gather


def kernel(x, word_embeddings):
    n_idx = x.size
    idx2d = x.reshape(-1, _IDXW).astype(jnp.int32)
    out = _make_gather(word_embeddings.shape[0], n_idx)(word_embeddings, idx2d)
    return out.reshape(x.shape + (EMBED,))


# speedup vs baseline: 1.2447x; 1.2229x over previous
import functools

import jax
import jax.numpy as jnp
from jax import lax
from jax.experimental import pallas as pl
from jax.experimental.pallas import tpu as pltpu
from jax.experimental.pallas import tpu_sc as plsc

EMBED = 64
_PAD = 128
_IDXW = 128
_CHUNK = 256
_STREAMS = _CHUNK // _IDXW
_NSLOT = 3


@functools.lru_cache(maxsize=None)
def _make_gather(vocab: int, n_idx: int):
    info = plsc.get_sparse_core_info()
    nc, ns = info.num_cores, info.num_subcores
    nw = nc * ns
    assert n_idx % (nw * _CHUNK) == 0
    per_w = n_idx // nw
    n_chunks = per_w // _CHUNK
    idx_rows = per_w // _IDXW

    mesh = plsc.VectorSubcoreMesh(core_axis_name="c", subcore_axis_name="s")

    @functools.partial(
        pl.kernel,
        mesh=mesh,
        out_type=jax.ShapeDtypeStruct((n_idx, _PAD), jnp.float32),
        scratch_types=[
            pltpu.VMEM((idx_rows, _IDXW), jnp.int32),
            pltpu.VMEM((_NSLOT * _CHUNK, _PAD), jnp.float32),
            pltpu.SemaphoreType.DMA,
            pltpu.SemaphoreType.DMA,
        ],
    )
    def gather(table_hbm, idx_hbm, out_hbm, idx_v, rows_v, g_sem, w_sem):
        wid = lax.axis_index("s") * nc + lax.axis_index("c")
        out_base = wid * per_w

        pltpu.sync_copy(idx_hbm.at[pl.ds(wid * idx_rows, idx_rows)], idx_v)

        def fire_gathers(g, slot):
            for j in range(_STREAMS):
                pltpu.async_copy(
                    table_hbm.at[idx_v.at[g * _STREAMS + j]],
                    rows_v.at[pl.ds(slot * _CHUNK + j * _IDXW, _IDXW)],
                    g_sem,
                )

        def drain_gathers():
            pltpu.make_async_copy(
                table_hbm.at[pl.ds(0, _CHUNK)],
                rows_v.at[pl.ds(0, _CHUNK)],
                g_sem,
            ).wait()

        def drain_write():
            pltpu.make_async_copy(
                rows_v.at[pl.ds(0, _CHUNK)],
                out_hbm.at[pl.ds(out_base, _CHUNK)],
                w_sem,
            ).wait()

        fire_gathers(0, 0)

        def step(g, carry):
            nxt_slot = lax.rem(g + 1, _NSLOT)

            @pl.when(g >= _NSLOT - 1)
            def _():
                drain_write()

            @pl.when(g + 1 < n_chunks)
            def _():
                fire_gathers(g + 1, nxt_slot)

            drain_gathers()
            pltpu.async_copy(
                rows_v.at[pl.ds(lax.rem(g, _NSLOT) * _CHUNK, _CHUNK)],
                out_hbm.at[pl.ds(out_base + g * _CHUNK, _CHUNK)],
                w_sem,
            )
            return carry

        lax.fori_loop(0, n_chunks, step, 0)
        for _ in range(min(_NSLOT - 1, n_chunks)):
            drain_write()

    return gather


def kernel(x, word_embeddings):
    n_idx = x.size
    tbl = jnp.pad(word_embeddings, ((0, 0), (0, _PAD - EMBED)))
    idx2d = x.reshape(-1, _IDXW).astype(jnp.int32)
    out = _make_gather(word_embeddings.shape[0], n_idx)(tbl, idx2d)
    return out[:, :EMBED].reshape(x.shape + (EMBED,))
